# baseline (speedup 1.0000x reference)
import functools

import jax
import jax.numpy as jnp
from jax import lax
from jax.experimental import pallas as pl
from jax.experimental.pallas import tpu as pltpu, tpu_sc as plsc

_N = 100000
_D = 128
_R = 4
_E = 150000
_B = 16384
_F = 11

_EP = 155648
_EPW = _EP // 16

_C = 12672
_NCH = 8
_NP = _C * _NCH
_ACC_ROWS = _C + 512
_SEG = 1216
_G = 64

_NZ = 101000
_BF = _B * _F
_BFW = _BF // 32
_GB = 32
_GBF = _GB * _F

_DEG_ROWS = 800
_NPD = _DEG_ROWS * 128

_mesh = plsc.VectorSubcoreMesh(core_axis_name="c", subcore_axis_name="s")
_sc_params = pltpu.CompilerParams(needs_layout_passes=False)


@functools.partial(
    pl.kernel,
    out_type=jax.ShapeDtypeStruct((32 * _DEG_ROWS, 128), jnp.float32),
    mesh=_mesh,
    compiler_params=_sc_params,
    scratch_types=[
        pltpu.VMEM((_DEG_ROWS, 128), jnp.float32),
        pltpu.VMEM((_EPW,), jnp.int32),
    ],
)
def _deg_kernel(eidx_hbm, out_hbm, acc_v, stage_v):
  c = lax.axis_index("c")
  s = lax.axis_index("s")
  pair = 4 * c + s // 4
  quarter = s % 4
  zeros16 = jnp.zeros((16,), jnp.float32)
  ones16 = jnp.ones((16,), jnp.float32)

  def _z(i, _):
    for j in range(8):
      acc_v[i, pl.ds(j * 16, 16)] = zeros16
    return 0
  lax.fori_loop(0, _DEG_ROWS, _z, 0)

  nq = _EP // 4
  for st in range(4):
    pltpu.sync_copy(
        eidx_hbm.at[pl.ds(pair * _EP + quarter * nq + st * _EPW, _EPW)],
        stage_v)

    def _h(j, _):
      idx = stage_v[pl.ds(j * 16, 16)]
      mk = idx >= 0
      plsc.addupdate_scatter(acc_v, [idx >> 7, idx & 127], ones16, mask=mk)
      return 0
    lax.fori_loop(0, _EPW // 16, _h, 0)

  pltpu.sync_copy(
      acc_v, out_hbm.at[pl.ds((pair * 4 + quarter) * _DEG_ROWS, _DEG_ROWS)])


@functools.partial(
    pl.kernel,
    out_type=jax.ShapeDtypeStruct((_R * _NP, 128), jnp.float32),
    mesh=_mesh,
    compiler_params=_sc_params,
    scratch_types=[
        pltpu.VMEM((2, 2 * _SEG), jnp.int32),
        pltpu.VMEM((_SEG + _G,), jnp.int32),
        pltpu.VMEM(((_SEG + _G) // _G, _G), jnp.int32),
        pltpu.VMEM((2, _G, 128), jnp.float32),
        pltpu.VMEM_SHARED((_ACC_ROWS, 128), jnp.float32),
        pltpu.SemaphoreType.DMA,
        pltpu.SemaphoreType.DMA,
        pltpu.SemaphoreType.DMA,
    ],
)
def _scatter_kernel(esd_hbm, zrows_hbm, mflat_hbm, agg_hbm,
                    esd_v, csrc_v, cldst_v, rows_v, acc_sh, sem, sem2, sem3):
  c = lax.axis_index("c")
  s = lax.axis_index("s")
  lane = lax.iota(jnp.int32, 16)
  zw = _ACC_ROWS // 16
  fw = _C // 16
  nseg = _EPW // _SEG

  def _stage(r, seg):
    base = (((r * 16 + s) * nseg) + seg) * (2 * _SEG)
    return pltpu.async_copy(esd_hbm.at[pl.ds(base, 2 * _SEG)],
                            esd_v.at[seg % 2], sem2)

  def _flush_slices(r, lo):
    return (acc_sh.at[pl.ds(s * fw, fw)],
            agg_hbm.at[pl.ds(r * _NP + lo + s * fw, fw)])

  def _chunk_rel(tr, _):
    t = tr // _R
    r = tr % _R
    ck = 2 * t + c
    lo = ck * _C

    st = _stage(r, 0)
    trp = jnp.maximum(tr - 1, 0)
    rp = trp % _R
    lop = (2 * (trp // _R) + c) * _C

    @pl.when(tr > 0)
    def _():
      srcp_, dstp_ = _flush_slices(rp, lop)
      pltpu.make_async_copy(srcp_, dstp_, sem3).wait()
    plsc.subcore_barrier()

    zeros16 = jnp.zeros((16,), jnp.float32)

    def _zb(i, _):
      for j in range(8):
        rows_v[0, i, pl.ds(j * 16, 16)] = zeros16
      return 0
    lax.fori_loop(0, _G, _zb, 0)
    nz = zw // _G
    for z in range(nz):
      pltpu.async_copy(rows_v.at[0], acc_sh.at[pl.ds(s * zw + z * _G, _G)],
                       sem)
    pltpu.async_copy(rows_v.at[0, pl.ds(0, zw - nz * _G)],
                     acc_sh.at[pl.ds(s * zw + nz * _G, zw - nz * _G)], sem)

    def _zdrain():
      for z in range(nz):
        pltpu.make_async_copy(
            rows_v.at[0], acc_sh.at[pl.ds(s * zw + z * _G, _G)], sem).wait()
      pltpu.make_async_copy(
          rows_v.at[0, pl.ds(0, zw - nz * _G)],
          acc_sh.at[pl.ds(s * zw + nz * _G, zw - nz * _G)], sem).wait()
      plsc.subcore_barrier()

    for seg in range(nseg):
      st.wait()
      if seg + 1 < nseg:
        st = _stage(r, seg + 1)
      slot = seg % 2

      def _vec(j, k):
        poff = k
        tot = None
        for u in range(4):
          src = esd_v[slot, pl.ds(j * 64 + u * 16, 16)]
          d = esd_v[slot, pl.ds(_SEG + j * 64 + u * 16, 16)]
          mk = (d >= lo) & (d < lo + _C)
          mki = jnp.where(mk, 1, 0)
          pos = poff + plsc.cumsum(mki) - 1
          plsc.store_scatter(csrc_v, [pos], src, mask=mk)
          plsc.store_scatter(cldst_v, [pos >> 6, pos & 63],
                             d - lo, mask=mk)
          cnt = plsc.all_reduce_population_count(mk)
          poff = poff + cnt
          tot = mki if tot is None else tot + mki
        return k + jnp.sum(tot)
      k = lax.fori_loop(0, _SEG // 64, _vec, 0)

      kp = (k + (_G - 1)) & (-_G)
      for j2 in range(_G // 16):
        pos2 = k + j2 * 16 + lane
        mk2 = pos2 < kp
        dmy = (lane * 61 + j2 * 16) & 1023
        plsc.store_scatter(csrc_v, [pos2], dmy, mask=mk2)
        plsc.store_scatter(cldst_v, [pos2 >> 6, pos2 & 63],
                           _C + ((lane + j2 * 16) & 511), mask=mk2)

      if seg == 0:
        _zdrain()

      nb = kp >> 6

      @pl.when(nb > 0)
      def _():
        pltpu.async_copy(mflat_hbm.at[csrc_v.at[pl.ds(0, _G)]],
                         rows_v.at[0], sem)

      def _fire(b, _):
        @pl.when(b + 1 < nb)
        def _():
          pltpu.async_copy(mflat_hbm.at[csrc_v.at[pl.ds((b + 1) * _G, _G)]],
                           rows_v.at[(b + 1) & 1], sem)
        pltpu.make_async_copy(mflat_hbm.at[csrc_v.at[pl.ds(b * _G, _G)]],
                              rows_v.at[b & 1], sem).wait()
        pltpu.sync_copy(rows_v.at[b & 1], acc_sh.at[cldst_v.at[b]], add=True)
        return 0
      lax.fori_loop(0, nb, _fire, 0)

    plsc.subcore_barrier()
    fsrc, fdst = _flush_slices(r, lo)
    pltpu.async_copy(fsrc, fdst, sem3)
    return 0

  npair = (_NCH // 2) * _R
  lax.fori_loop(0, npair, _chunk_rel, 0)
  fsrc, fdst = _flush_slices(_R - 1, (2 * ((npair - 1) // _R) + c) * _C)
  pltpu.make_async_copy(fsrc, fdst, sem3).wait()


@functools.partial(
    pl.kernel,
    out_type=jax.ShapeDtypeStruct((_B, 128), jnp.float32),
    mesh=_mesh,
    compiler_params=_sc_params,
    scratch_types=[
        pltpu.VMEM((_GBF,), jnp.int32),
        pltpu.VMEM((_GBF, 128), jnp.float32),
        pltpu.VMEM((_GB, 128), jnp.float32),
        pltpu.SemaphoreType.DMA,
    ],
)
def _gmean_kernel(fidx_hbm, h2_hbm, fsum_hbm, fidx_v, rows_v, out_v, sem):
  c = lax.axis_index("c")
  s = lax.axis_index("s")
  w = s * 2 + c

  def _grp(g, _):
    base = w * _BFW + g * _GBF
    pltpu.sync_copy(fidx_hbm.at[pl.ds(base, _GBF)], fidx_v)
    pltpu.async_copy(h2_hbm.at[fidx_v], rows_v, sem).wait()

    def _row(b, _):
      for j in range(8):
        acc = rows_v[b * _F, pl.ds(j * 16, 16)]
        for f in range(1, _F):
          acc = acc + rows_v[b * _F + f, pl.ds(j * 16, 16)]
        out_v[b, pl.ds(j * 16, 16)] = acc
      return 0
    lax.fori_loop(0, _GB, _row, 0)
    pltpu.sync_copy(out_v, fsum_hbm.at[pl.ds(w * (_BFW // _F) + g * _GB, _GB)])
    return 0
  lax.fori_loop(0, _BFW // _GBF, _grp, 0)


_BN = 1000


def _degsum(deg, pair):
  d = deg[0, pair * 4, :]
  for q in range(1, 4):
    d = d + deg[0, pair * 4 + q, :]
  return lax.rsqrt(jnp.maximum(d, 1.0))


def _bdot(a, b):
  return jnp.dot(a.astype(jnp.bfloat16), b.astype(jnp.bfloat16),
                 preferred_element_type=jnp.float32)


def _tc1_body(nf, w_in, b_in, w1, deg, m_out):
  i = pl.program_id(0)
  h = jnp.maximum(_bdot(nf[...], w_in[...]) + b_in[...], 0.0)
  for r in range(_R):
    so = _degsum(deg, r)
    m_out[r] = _bdot(h * so[:, None], w1[r])


def _tc2_body(agg, deg, b1s, w2, m_out):
  i = pl.program_id(0)
  acc = b1s[...]
  for r in range(_R):
    si = _degsum(deg, 4 + r)
    acc = acc + agg[r] * si[:, None]
  h1 = jnp.maximum(acc, 0.0)
  for r in range(_R):
    so = _degsum(deg, r)
    m_out[r] = _bdot(h1 * so[:, None], w2[r])


def _tc3_body(agg, deg, b2s, h2_out):
  i = pl.program_id(0)
  ii = lax.min(i, _N // _BN - 1)

  @pl.when(i < _N // _BN)
  def _():
    acc = b2s[...]
    for r in range(_R):
      si = _degsum(deg, 4 + r)
      acc = acc + agg[r] * si[:, None]
    h2_out[...] = acc

  @pl.when(i >= _N // _BN)
  def _():
    h2_out[...] = jnp.zeros((_BN, 128), jnp.float32)


def _tc4_body(fsum, mf, wm1, bm1, wm2, bm2, wm3, bm3, out):
  num = jnp.maximum(jnp.sum(mf[...], axis=1, keepdims=True), 1.0)
  z = fsum[...] / num
  z = jnp.maximum(
      jnp.dot(z, wm1[...], preferred_element_type=jnp.float32) + bm1[...], 0.0)
  z = jnp.maximum(
      jnp.dot(z, wm2[...], preferred_element_type=jnp.float32) + bm2[...], 0.0)
  out[...] = jnp.dot(z, wm3[...], preferred_element_type=jnp.float32) + bm3[...]


def _whole(shape):
  return pl.BlockSpec(shape, lambda i: tuple(0 for _ in shape))


def kernel(node_feat, W_in, b_in, W1, b1, W2, b2, Wm1, bm1, Wm2, bm2, Wm3,
           bm3, edge_index, indices, mask):
  f32 = jnp.float32
  src = edge_index[:, 0, :]
  dst = edge_index[:, 1, :]
  pad = _EP - _E
  pad_rows = (jnp.arange(pad, dtype=jnp.int32) * 97) % _N
  gsrc = jnp.concatenate(
      [src + (jnp.arange(_R, dtype=jnp.int32) * _N)[:, None],
       jnp.broadcast_to(pad_rows, (_R, pad))], axis=1)
  dstp = jnp.concatenate(
      [dst, jnp.full((_R, pad), -1, jnp.int32)], axis=1)
  esd = jnp.stack(
      [gsrc.reshape(_R, 16, _EPW // _SEG, _SEG),
       dstp.reshape(_R, 16, _EPW // _SEG, _SEG)],
      axis=3).reshape(_R * _EP * 2)
  zrows = jnp.zeros((_ACC_ROWS // 16, _D), f32)
  gsrc = gsrc.reshape(_R * _EP)
  dstp = dstp.reshape(_R * _EP)
  srcp = jnp.concatenate(
      [src, jnp.full((_R, pad), -1, jnp.int32)], axis=1).reshape(_R * _EP)
  eidx = jnp.concatenate([
      srcp, jnp.where(dstp >= 0, dstp, -1)], axis=0)

  flat_pos = jnp.arange(_BF, dtype=jnp.int32).reshape(_B, _F)
  fidx = jnp.where(mask > 0, indices, _N + (flat_pos % 1000)).reshape(_BF)

  b_in2 = b_in.reshape(1, _D)
  b1s = jnp.sum(b1, axis=0).reshape(1, _D)
  b2s = jnp.sum(b2, axis=0).reshape(1, _D)
  bm1_2 = bm1.reshape(1, _D)
  bm2_2 = bm2.reshape(1, _D)
  bm3_2 = bm3.reshape(1, 2)

  deg2d = _deg_kernel(eidx)
  deg = deg2d.reshape(32, _NPD)[:, :_N].reshape(32, _N // _BN, _BN).transpose(1, 0, 2)

  grid1 = (_N // _BN,)
  m1 = pl.pallas_call(
      _tc1_body,
      grid=grid1,
      in_specs=[
          pl.BlockSpec((_BN, _D), lambda i: (i, 0)),
          _whole((_D, _D)),
          _whole((1, _D)),
          _whole((_R, _D, _D)),
          pl.BlockSpec((1, 32, _BN), lambda i: (i, 0, 0)),
      ],
      out_specs=pl.BlockSpec((_R, _BN, _D), lambda i: (0, i, 0)),
      out_shape=jax.ShapeDtypeStruct((_R, _N, _D), f32),
  )(node_feat, W_in, b_in2, W1, deg)

  agg1 = _scatter_kernel(esd, zrows, m1.reshape(_R * _N, _D))
  agg1 = agg1.reshape(_R, _NP, _D)

  m2 = pl.pallas_call(
      _tc2_body,
      grid=grid1,
      in_specs=[
          pl.BlockSpec((_R, _BN, _D), lambda i: (0, i, 0)),
          pl.BlockSpec((1, 32, _BN), lambda i: (i, 0, 0)),
          _whole((1, _D)),
          _whole((_R, _D, _D)),
      ],
      out_specs=pl.BlockSpec((_R, _BN, _D), lambda i: (0, i, 0)),
      out_shape=jax.ShapeDtypeStruct((_R, _N, _D), f32),
  )(agg1, deg, b1s, W2)

  agg2 = _scatter_kernel(esd, zrows, m2.reshape(_R * _N, _D))
  agg2 = agg2.reshape(_R, _NP, _D)

  h2 = pl.pallas_call(
      _tc3_body,
      grid=(_NZ // _BN,),
      in_specs=[
          pl.BlockSpec((_R, _BN, _D),
                       lambda i: (0, lax.min(i, _N // _BN - 1), 0)),
          pl.BlockSpec((1, 32, _BN), lambda i: (i, 0, 0)),
          _whole((1, _D)),
      ],
      out_specs=pl.BlockSpec((_BN, _D), lambda i: (i, 0)),
      out_shape=jax.ShapeDtypeStruct((_NZ, _D), f32),
  )(agg2, deg, b2s)

  fsum = _gmean_kernel(fidx, h2)

  out = pl.pallas_call(
      _tc4_body,
      grid=(_B // 2048,),
      in_specs=[
          pl.BlockSpec((2048, _D), lambda i: (i, 0)),
          pl.BlockSpec((2048, _F), lambda i: (i, 0)),
          _whole((_D, _D)),
          _whole((1, _D)),
          _whole((_D, _D)),
          _whole((1, _D)),
          _whole((_D, 2)),
          _whole((1, 2)),
      ],
      out_specs=pl.BlockSpec((2048, 2), lambda i: (i, 0)),
      out_shape=jax.ShapeDtypeStruct((_B, 2), f32),
  )(fsum, mask.astype(f32), Wm1, bm1_2, Wm2, bm2_2, Wm3, bm3_2)
  return out


# speedup vs baseline: 1.0006x; 1.0006x over previous
import functools

import jax
import jax.numpy as jnp
from jax import lax
from jax.experimental import pallas as pl
from jax.experimental.pallas import tpu as pltpu, tpu_sc as plsc

_N = 100000
_D = 128
_R = 4
_E = 150000
_B = 16384
_F = 11

_EP = 155648
_EPW = _EP // 16

_C = 12672
_NCH = 8
_NP = _C * _NCH
_ACC_ROWS = _C + 512
_SEG = 1216
_G = 64

_NZ = 101000
_BF = _B * _F
_BFW = _BF // 32
_GB = 32
_GBF = _GB * _F

_DEG_ROWS = 800
_NPD = _DEG_ROWS * 128

_mesh = plsc.VectorSubcoreMesh(core_axis_name="c", subcore_axis_name="s")
_sc_params = pltpu.CompilerParams(needs_layout_passes=False)


@functools.partial(
    pl.kernel,
    out_type=jax.ShapeDtypeStruct((32 * _DEG_ROWS, 128), jnp.float32),
    mesh=_mesh,
    compiler_params=_sc_params,
    scratch_types=[
        pltpu.VMEM((_DEG_ROWS, 128), jnp.float32),
        pltpu.VMEM((_EPW,), jnp.int32),
    ],
)
def _deg_kernel(eidx_hbm, out_hbm, acc_v, stage_v):
  c = lax.axis_index("c")
  s = lax.axis_index("s")
  pair = 4 * c + s // 4
  quarter = s % 4
  zeros16 = jnp.zeros((16,), jnp.float32)
  ones16 = jnp.ones((16,), jnp.float32)

  def _z(i, _):
    for j in range(8):
      acc_v[i, pl.ds(j * 16, 16)] = zeros16
    return 0
  lax.fori_loop(0, _DEG_ROWS, _z, 0)

  nq = _EP // 4
  for st in range(4):
    pltpu.sync_copy(
        eidx_hbm.at[pl.ds(pair * _EP + quarter * nq + st * _EPW, _EPW)],
        stage_v)

    def _h(j, _):
      idx = stage_v[pl.ds(j * 16, 16)]
      mk = idx >= 0
      plsc.addupdate_scatter(acc_v, [idx >> 7, idx & 127], ones16, mask=mk)
      return 0
    lax.fori_loop(0, _EPW // 16, _h, 0)

  pltpu.sync_copy(
      acc_v, out_hbm.at[pl.ds((pair * 4 + quarter) * _DEG_ROWS, _DEG_ROWS)])


@functools.partial(
    pl.kernel,
    out_type=jax.ShapeDtypeStruct((_R * _NP, 128), jnp.float32),
    mesh=_mesh,
    compiler_params=_sc_params,
    scratch_types=[
        pltpu.VMEM((2, 2 * _SEG), jnp.int32),
        pltpu.VMEM((_SEG + _G,), jnp.int32),
        pltpu.VMEM(((_SEG + _G) // _G, _G), jnp.int32),
        pltpu.VMEM((2, _G, 128), jnp.float32),
        pltpu.VMEM_SHARED((_ACC_ROWS, 128), jnp.float32),
        pltpu.SemaphoreType.DMA,
        pltpu.SemaphoreType.DMA,
        pltpu.SemaphoreType.DMA,
    ],
)
def _scatter_kernel(esd_hbm, zrows_hbm, mflat_hbm, agg_hbm,
                    esd_v, csrc_v, cldst_v, rows_v, acc_sh, sem, sem2, sem3):
  c = lax.axis_index("c")
  s = lax.axis_index("s")
  lane = lax.iota(jnp.int32, 16)
  zw = _ACC_ROWS // 16
  fw = _C // 16
  nseg = _EPW // _SEG

  def _stage(r, seg):
    base = (((r * 16 + s) * nseg) + seg) * (2 * _SEG)
    return pltpu.async_copy(esd_hbm.at[pl.ds(base, 2 * _SEG)],
                            esd_v.at[seg % 2], sem2)

  def _flush_slices(r, lo):
    return (acc_sh.at[pl.ds(s * fw, fw)],
            agg_hbm.at[pl.ds(r * _NP + lo + s * fw, fw)])

  def _chunk_rel(tr, _):
    t = tr // _R
    r = tr % _R
    ck = 2 * t + c
    lo = ck * _C

    st = _stage(r, 0)
    trp = jnp.maximum(tr - 1, 0)
    rp = trp % _R
    lop = (2 * (trp // _R) + c) * _C

    @pl.when(tr > 0)
    def _():
      srcp_, dstp_ = _flush_slices(rp, lop)
      pltpu.make_async_copy(srcp_, dstp_, sem3).wait()
    plsc.subcore_barrier()

    zeros16 = jnp.zeros((16,), jnp.float32)

    def _zb(i, _):
      for j in range(8):
        rows_v[0, i, pl.ds(j * 16, 16)] = zeros16
      return 0
    lax.fori_loop(0, _G, _zb, 0)
    nz = zw // _G
    for z in range(nz):
      pltpu.async_copy(rows_v.at[0], acc_sh.at[pl.ds(s * zw + z * _G, _G)],
                       sem)
    pltpu.async_copy(rows_v.at[0, pl.ds(0, zw - nz * _G)],
                     acc_sh.at[pl.ds(s * zw + nz * _G, zw - nz * _G)], sem)

    def _zdrain():
      for z in range(nz):
        pltpu.make_async_copy(
            rows_v.at[0], acc_sh.at[pl.ds(s * zw + z * _G, _G)], sem).wait()
      pltpu.make_async_copy(
          rows_v.at[0, pl.ds(0, zw - nz * _G)],
          acc_sh.at[pl.ds(s * zw + nz * _G, zw - nz * _G)], sem).wait()
      plsc.subcore_barrier()

    for seg in range(nseg):
      st.wait()
      if seg + 1 < nseg:
        st = _stage(r, seg + 1)
      slot = seg % 2

      def _vec(j, k):
        poff = k
        tot = None
        for u in range(4):
          src = esd_v[slot, pl.ds(j * 64 + u * 16, 16)]
          d = esd_v[slot, pl.ds(_SEG + j * 64 + u * 16, 16)]
          mk = (d >= lo) & (d < lo + _C)
          mki = jnp.where(mk, 1, 0)
          pos = poff + plsc.cumsum(mki) - 1
          plsc.store_scatter(csrc_v, [pos], src, mask=mk)
          plsc.store_scatter(cldst_v, [pos >> 6, pos & 63],
                             d - lo, mask=mk)
          cnt = plsc.all_reduce_population_count(mk)
          poff = poff + cnt
          tot = mki if tot is None else tot + mki
        return k + jnp.sum(tot)
      k = lax.fori_loop(0, _SEG // 64, _vec, 0)

      kp = (k + (_G - 1)) & (-_G)
      for j2 in range(_G // 16):
        pos2 = k + j2 * 16 + lane
        mk2 = pos2 < kp
        dmy = (lane * 61 + j2 * 16) & 1023
        plsc.store_scatter(csrc_v, [pos2], dmy, mask=mk2)
        plsc.store_scatter(cldst_v, [pos2 >> 6, pos2 & 63],
                           _C + ((lane + j2 * 16) & 511), mask=mk2)

      if seg == 0:
        _zdrain()

      nb = kp >> 6

      @pl.when(nb > 0)
      def _():
        pltpu.async_copy(mflat_hbm.at[csrc_v.at[pl.ds(0, _G)]],
                         rows_v.at[0], sem)

      def _fire(b, _):
        @pl.when(b + 1 < nb)
        def _():
          pltpu.async_copy(mflat_hbm.at[csrc_v.at[pl.ds((b + 1) * _G, _G)]],
                           rows_v.at[(b + 1) & 1], sem)
        pltpu.make_async_copy(mflat_hbm.at[csrc_v.at[pl.ds(b * _G, _G)]],
                              rows_v.at[b & 1], sem).wait()
        pltpu.sync_copy(rows_v.at[b & 1], acc_sh.at[cldst_v.at[b]], add=True)
        return 0
      lax.fori_loop(0, nb, _fire, 0)

    plsc.subcore_barrier()
    fsrc, fdst = _flush_slices(r, lo)
    pltpu.async_copy(fsrc, fdst, sem3)
    return 0

  npair = (_NCH // 2) * _R
  lax.fori_loop(0, npair, _chunk_rel, 0)
  fsrc, fdst = _flush_slices(_R - 1, (2 * ((npair - 1) // _R) + c) * _C)
  pltpu.make_async_copy(fsrc, fdst, sem3).wait()


@functools.partial(
    pl.kernel,
    out_type=jax.ShapeDtypeStruct((_B, 128), jnp.float32),
    mesh=_mesh,
    compiler_params=_sc_params,
    scratch_types=[
        pltpu.VMEM((_GBF,), jnp.int32),
        pltpu.VMEM((_GBF, 128), jnp.float32),
        pltpu.VMEM((_GB, 128), jnp.float32),
        pltpu.SemaphoreType.DMA,
    ],
)
def _gmean_kernel(fidx_hbm, h2_hbm, fsum_hbm, fidx_v, rows_v, out_v, sem):
  c = lax.axis_index("c")
  s = lax.axis_index("s")
  w = s * 2 + c

  def _grp(g, _):
    base = w * _BFW + g * _GBF
    pltpu.sync_copy(fidx_hbm.at[pl.ds(base, _GBF)], fidx_v)
    pltpu.async_copy(h2_hbm.at[fidx_v], rows_v, sem).wait()

    def _row(b, _):
      for j in range(8):
        acc = rows_v[b * _F, pl.ds(j * 16, 16)]
        for f in range(1, _F):
          acc = acc + rows_v[b * _F + f, pl.ds(j * 16, 16)]
        out_v[b, pl.ds(j * 16, 16)] = acc
      return 0
    lax.fori_loop(0, _GB, _row, 0)
    pltpu.sync_copy(out_v, fsum_hbm.at[pl.ds(w * (_BFW // _F) + g * _GB, _GB)])
    return 0
  lax.fori_loop(0, _BFW // _GBF, _grp, 0)


_BN = 1000


def _degsum(deg, pair):
  d = deg[0, pair * 4, :]
  for q in range(1, 4):
    d = d + deg[0, pair * 4 + q, :]
  return lax.rsqrt(jnp.maximum(d, 1.0))


def _tc1_body(nf, w_in, b_in, w1, deg, m_out):
  i = pl.program_id(0)
  h = jnp.maximum(
      jnp.dot(nf[...], w_in[...], preferred_element_type=jnp.float32)
      + b_in[...], 0.0)
  for r in range(_R):
    so = _degsum(deg, r)
    m_out[r] = jnp.dot(h * so[:, None], w1[r],
                       preferred_element_type=jnp.float32)


def _tc2_body(agg, deg, b1s, w2, m_out):
  i = pl.program_id(0)
  acc = b1s[...]
  for r in range(_R):
    si = _degsum(deg, 4 + r)
    acc = acc + agg[r] * si[:, None]
  h1 = jnp.maximum(acc, 0.0)
  for r in range(_R):
    so = _degsum(deg, r)
    m_out[r] = jnp.dot(h1 * so[:, None], w2[r],
                       preferred_element_type=jnp.float32)


def _tc3_body(agg, deg, b2s, h2_out):
  i = pl.program_id(0)
  ii = lax.min(i, _N // _BN - 1)

  @pl.when(i < _N // _BN)
  def _():
    acc = b2s[...]
    for r in range(_R):
      si = _degsum(deg, 4 + r)
      acc = acc + agg[r] * si[:, None]
    h2_out[...] = acc

  @pl.when(i >= _N // _BN)
  def _():
    h2_out[...] = jnp.zeros((_BN, 128), jnp.float32)


def _tc4_body(fsum, mf, wm1, bm1, wm2, bm2, wm3, bm3, out):
  num = jnp.maximum(jnp.sum(mf[...], axis=1, keepdims=True), 1.0)
  z = fsum[...] / num
  z = jnp.maximum(
      jnp.dot(z, wm1[...], preferred_element_type=jnp.float32) + bm1[...], 0.0)
  z = jnp.maximum(
      jnp.dot(z, wm2[...], preferred_element_type=jnp.float32) + bm2[...], 0.0)
  out[...] = jnp.dot(z, wm3[...], preferred_element_type=jnp.float32) + bm3[...]


def _whole(shape):
  return pl.BlockSpec(shape, lambda i: tuple(0 for _ in shape))


def kernel(node_feat, W_in, b_in, W1, b1, W2, b2, Wm1, bm1, Wm2, bm2, Wm3,
           bm3, edge_index, indices, mask):
  f32 = jnp.float32
  src = edge_index[:, 0, :]
  dst = edge_index[:, 1, :]
  pad = _EP - _E
  pad_rows = (jnp.arange(pad, dtype=jnp.int32) * 97) % _N
  gsrc = jnp.concatenate(
      [src + (jnp.arange(_R, dtype=jnp.int32) * _N)[:, None],
       jnp.broadcast_to(pad_rows, (_R, pad))], axis=1)
  dstp = jnp.concatenate(
      [dst, jnp.full((_R, pad), -1, jnp.int32)], axis=1)
  esd = jnp.stack(
      [gsrc.reshape(_R, 16, _EPW // _SEG, _SEG),
       dstp.reshape(_R, 16, _EPW // _SEG, _SEG)],
      axis=3).reshape(_R * _EP * 2)
  zrows = jnp.zeros((_ACC_ROWS // 16, _D), f32)
  gsrc = gsrc.reshape(_R * _EP)
  dstp = dstp.reshape(_R * _EP)
  srcp = jnp.concatenate(
      [src, jnp.full((_R, pad), -1, jnp.int32)], axis=1).reshape(_R * _EP)
  eidx = jnp.concatenate([
      srcp, jnp.where(dstp >= 0, dstp, -1)], axis=0)

  flat_pos = jnp.arange(_BF, dtype=jnp.int32).reshape(_B, _F)
  fidx = jnp.where(mask > 0, indices, _N + (flat_pos % 1000)).reshape(_BF)

  b_in2 = b_in.reshape(1, _D)
  b1s = jnp.sum(b1, axis=0).reshape(1, _D)
  b2s = jnp.sum(b2, axis=0).reshape(1, _D)
  bm1_2 = bm1.reshape(1, _D)
  bm2_2 = bm2.reshape(1, _D)
  bm3_2 = bm3.reshape(1, 2)

  deg2d = _deg_kernel(eidx)
  deg = deg2d.reshape(32, _NPD)[:, :_N].reshape(32, _N // _BN, _BN).transpose(1, 0, 2)

  grid1 = (_N // _BN,)
  m1 = pl.pallas_call(
      _tc1_body,
      grid=grid1,
      in_specs=[
          pl.BlockSpec((_BN, _D), lambda i: (i, 0)),
          _whole((_D, _D)),
          _whole((1, _D)),
          _whole((_R, _D, _D)),
          pl.BlockSpec((1, 32, _BN), lambda i: (i, 0, 0)),
      ],
      out_specs=pl.BlockSpec((_R, _BN, _D), lambda i: (0, i, 0)),
      out_shape=jax.ShapeDtypeStruct((_R, _N, _D), f32),
  )(node_feat, W_in, b_in2, W1, deg)

  agg1 = _scatter_kernel(esd, zrows, m1.reshape(_R * _N, _D))
  agg1 = agg1.reshape(_R, _NP, _D)

  m2 = pl.pallas_call(
      _tc2_body,
      grid=grid1,
      in_specs=[
          pl.BlockSpec((_R, _BN, _D), lambda i: (0, i, 0)),
          pl.BlockSpec((1, 32, _BN), lambda i: (i, 0, 0)),
          _whole((1, _D)),
          _whole((_R, _D, _D)),
      ],
      out_specs=pl.BlockSpec((_R, _BN, _D), lambda i: (0, i, 0)),
      out_shape=jax.ShapeDtypeStruct((_R, _N, _D), f32),
  )(agg1, deg, b1s, W2)

  agg2 = _scatter_kernel(esd, zrows, m2.reshape(_R * _N, _D))
  agg2 = agg2.reshape(_R, _NP, _D)

  h2 = pl.pallas_call(
      _tc3_body,
      grid=(_NZ // _BN,),
      in_specs=[
          pl.BlockSpec((_R, _BN, _D),
                       lambda i: (0, lax.min(i, _N // _BN - 1), 0)),
          pl.BlockSpec((1, 32, _BN), lambda i: (i, 0, 0)),
          _whole((1, _D)),
      ],
      out_specs=pl.BlockSpec((_BN, _D), lambda i: (i, 0)),
      out_shape=jax.ShapeDtypeStruct((_NZ, _D), f32),
  )(agg2, deg, b2s)

  fsum = _gmean_kernel(fidx, h2)

  out = pl.pallas_call(
      _tc4_body,
      grid=(_B // 2048,),
      in_specs=[
          pl.BlockSpec((2048, _D), lambda i: (i, 0)),
          pl.BlockSpec((2048, _F), lambda i: (i, 0)),
          _whole((_D, _D)),
          _whole((1, _D)),
          _whole((_D, _D)),
          _whole((1, _D)),
          _whole((_D, 2)),
          _whole((1, 2)),
      ],
      out_specs=pl.BlockSpec((2048, 2), lambda i: (i, 0)),
      out_shape=jax.ShapeDtypeStruct((_B, 2), f32),
  )(fsum, mask.astype(f32), Wm1, bm1_2, Wm2, bm2_2, Wm3, bm3_2)
  return out
